# Initial kernel scaffold; baseline (speedup 1.0000x reference)
import jax
import jax.numpy as jnp
from jax.experimental import pallas as pl


def kernel(char_encoded, C_lengths, elmo_embeddings, glove_embeddings, onehot_embeddings, cemb, c_Wih, c_Whh, c_bih, c_bhh, w_Wih_f, w_Whh_f, w_bih_f, w_bhh_f, w_Wih_b, w_Whh_b, w_bih_b, w_bhh_b):
    raise NotImplementedError("write your pallas kernel here")


# speedup vs baseline: 4.3861x; 4.3861x over previous
import jax
import jax.numpy as jnp
from jax.experimental import pallas as pl
from jax.experimental.pallas import tpu as pltpu

_F32 = jnp.float32


def _char_body(ids_ref, idxm_ref, cemb_ref, wihT_ref, whhT_ref, bih_ref, bhh_ref,
               out_ref):
    WB, L = ids_ref.shape
    NC, CD = cemb_ref.shape
    E = jnp.dot(cemb_ref[...], wihT_ref[...], preferred_element_type=_F32)
    bih = bih_ref[...]
    bhh = bhh_ref[...]
    whhT = whhT_ref[...]
    idxm = idxm_ref[...]
    h = jnp.zeros((WB, CD), _F32)
    hsel = jnp.zeros((WB, CD), _F32)
    for t in range(L):
        ids_t = ids_ref[:, t:t + 1]
        oh = (ids_t == jax.lax.broadcasted_iota(jnp.int32, (WB, NC), 1)
              ).astype(_F32)
        gi = jnp.dot(oh, E, preferred_element_type=_F32) + bih
        gh = jnp.dot(h, whhT, preferred_element_type=_F32) + bhh
        r = jax.nn.sigmoid(gi[:, :CD] + gh[:, :CD])
        z = jax.nn.sigmoid(gi[:, CD:2 * CD] + gh[:, CD:2 * CD])
        n = jnp.tanh(gi[:, 2 * CD:] + r * gh[:, 2 * CD:])
        h = (1.0 - z) * n + z * h
        hsel = jnp.where(idxm == float(t), h, hsel)
    out_ref[...] = hsel


def _proj_body(e_ref, g_ref, c_ref, o_ref, we_ref, wg_ref, wc_ref, wo_ref,
               b_ref, out_ref):
    acc = jnp.dot(e_ref[...], we_ref[...], preferred_element_type=_F32)
    acc = acc + jnp.dot(g_ref[...], wg_ref[...], preferred_element_type=_F32)
    acc = acc + jnp.dot(c_ref[...], wc_ref[...], preferred_element_type=_F32)
    acc = acc + jnp.dot(o_ref[...], wo_ref[...], preferred_element_type=_F32)
    out_ref[...] = acc + b_ref[...]


def _rec_body(gif_ref, gib_ref, whf_ref, whb_ref, bhf_ref, bhb_ref,
              outf_ref, outb_ref, hf_ref, hb_ref):
    t = pl.program_id(0)

    @pl.when(t == 0)
    def _():
        hf_ref[...] = jnp.zeros_like(hf_ref)
        hb_ref[...] = jnp.zeros_like(hb_ref)

    H = whf_ref.shape[0]

    def step(gi, h, whhT, bhh):
        gh = jnp.dot(h, whhT, preferred_element_type=_F32) + bhh
        r = jax.nn.sigmoid(gi[:, :H] + gh[:, :H])
        z = jax.nn.sigmoid(gi[:, H:2 * H] + gh[:, H:2 * H])
        n = jnp.tanh(gi[:, 2 * H:] + r * gh[:, 2 * H:])
        return (1.0 - z) * n + z * h

    hf = step(gif_ref[0], hf_ref[...], whf_ref[...], bhf_ref[...])
    hf_ref[...] = hf
    outf_ref[0] = hf
    hb = step(gib_ref[0], hb_ref[...], whb_ref[...], bhb_ref[...])
    hb_ref[...] = hb
    outb_ref[0] = hb


def kernel(char_encoded, C_lengths, elmo_embeddings, glove_embeddings,
           onehot_embeddings, cemb, c_Wih, c_Whh, c_bih, c_bhh,
           w_Wih_f, w_Whh_f, w_bih_f, w_bhh_f,
           w_Wih_b, w_Whh_b, w_bih_b, w_bhh_b):
    NW, L = char_encoded.shape
    NC, CD = cemb.shape
    B, S, DE = elmo_embeddings.shape
    DG = glove_embeddings.shape[2]
    DO = onehot_embeddings.shape[2]
    H = w_Whh_f.shape[1]

    WB = 1024
    RB = 512

    idx = jnp.clip(C_lengths - 1, 0, L - 1).astype(_F32)
    idxm = jnp.broadcast_to(idx[:, None], (NW, CD))

    char_emb = pl.pallas_call(
        _char_body,
        grid=(NW // WB,),
        in_specs=[
            pl.BlockSpec((WB, L), lambda i: (i, 0)),
            pl.BlockSpec((WB, CD), lambda i: (i, 0)),
            pl.BlockSpec((NC, CD), lambda i: (0, 0)),
            pl.BlockSpec((CD, 3 * CD), lambda i: (0, 0)),
            pl.BlockSpec((CD, 3 * CD), lambda i: (0, 0)),
            pl.BlockSpec((1, 3 * CD), lambda i: (0, 0)),
            pl.BlockSpec((1, 3 * CD), lambda i: (0, 0)),
        ],
        out_specs=pl.BlockSpec((WB, CD), lambda i: (i, 0)),
        out_shape=jax.ShapeDtypeStruct((NW, CD), _F32),
    )(char_encoded.astype(jnp.int32), idxm, cemb, c_Wih.T, c_Whh.T,
      c_bih.reshape(1, -1), c_bhh.reshape(1, -1))

    WT = jnp.concatenate([w_Wih_f.T, w_Wih_b.T], axis=1)
    WT_e = WT[:DE]
    WT_g = WT[DE:DE + DG]
    WT_c = WT[DE + DG:DE + DG + CD]
    WT_o = WT[DE + DG + CD:]
    bi = jnp.concatenate([w_bih_f, w_bih_b]).reshape(1, -1)
    G = 6 * H

    gi = pl.pallas_call(
        _proj_body,
        grid=(NW // RB,),
        in_specs=[
            pl.BlockSpec((RB, DE), lambda i: (i, 0)),
            pl.BlockSpec((RB, DG), lambda i: (i, 0)),
            pl.BlockSpec((RB, CD), lambda i: (i, 0)),
            pl.BlockSpec((RB, DO), lambda i: (i, 0)),
            pl.BlockSpec((DE, G), lambda i: (0, 0)),
            pl.BlockSpec((DG, G), lambda i: (0, 0)),
            pl.BlockSpec((CD, G), lambda i: (0, 0)),
            pl.BlockSpec((DO, G), lambda i: (0, 0)),
            pl.BlockSpec((1, G), lambda i: (0, 0)),
        ],
        out_specs=pl.BlockSpec((RB, G), lambda i: (i, 0)),
        out_shape=jax.ShapeDtypeStruct((NW, G), _F32),
    )(elmo_embeddings.reshape(NW, DE), glove_embeddings.reshape(NW, DG),
      char_emb, onehot_embeddings.reshape(NW, DO),
      WT_e, WT_g, WT_c, WT_o, bi)

    gi3 = gi.reshape(B, S, G)

    hs_f, hs_b = pl.pallas_call(
        _rec_body,
        grid=(B,),
        in_specs=[
            pl.BlockSpec((1, S, 3 * H), lambda t: (t, 0, 0)),
            pl.BlockSpec((1, S, 3 * H), lambda t: (B - 1 - t, 0, 1)),
            pl.BlockSpec((H, 3 * H), lambda t: (0, 0)),
            pl.BlockSpec((H, 3 * H), lambda t: (0, 0)),
            pl.BlockSpec((1, 3 * H), lambda t: (0, 0)),
            pl.BlockSpec((1, 3 * H), lambda t: (0, 0)),
        ],
        out_specs=[
            pl.BlockSpec((1, S, H), lambda t: (t, 0, 0)),
            pl.BlockSpec((1, S, H), lambda t: (B - 1 - t, 0, 0)),
        ],
        out_shape=[jax.ShapeDtypeStruct((B, S, H), _F32),
                   jax.ShapeDtypeStruct((B, S, H), _F32)],
        scratch_shapes=[pltpu.VMEM((S, H), _F32), pltpu.VMEM((S, H), _F32)],
    )(gi3, gi3, w_Whh_f.T, w_Whh_b.T,
      w_bhh_f.reshape(1, -1), w_bhh_b.reshape(1, -1))

    return jnp.concatenate([hs_f, hs_b], axis=-1)


# speedup vs baseline: 4.4485x; 1.0142x over previous
import jax
import jax.numpy as jnp
from jax.experimental import pallas as pl
from jax.experimental.pallas import tpu as pltpu

_F32 = jnp.float32
_BF16 = jnp.bfloat16


def _char_body(ids_ref, idxm_ref, cemb_ref, wihT_ref, whhT_ref, bih_ref, bhh_ref,
               out_ref):
    WB, L = ids_ref.shape
    NC, CD = cemb_ref.shape
    E = jnp.dot(cemb_ref[...], wihT_ref[...],
                preferred_element_type=_F32).astype(_BF16)
    bih = bih_ref[...]
    bhh = bhh_ref[...]
    whhT = whhT_ref[...]
    idxm = idxm_ref[...]
    h = jnp.zeros((WB, CD), _F32)
    hsel = jnp.zeros((WB, CD), _F32)
    for t in range(L):
        ids_t = ids_ref[:, t:t + 1]
        oh = (ids_t == jax.lax.broadcasted_iota(jnp.int32, (WB, NC), 1)
              ).astype(_BF16)
        gi = jnp.dot(oh, E, preferred_element_type=_F32) + bih
        gh = jnp.dot(h.astype(_BF16), whhT, preferred_element_type=_F32) + bhh
        r = jax.nn.sigmoid(gi[:, :CD] + gh[:, :CD])
        z = jax.nn.sigmoid(gi[:, CD:2 * CD] + gh[:, CD:2 * CD])
        n = jnp.tanh(gi[:, 2 * CD:] + r * gh[:, 2 * CD:])
        h = (1.0 - z) * n + z * h
        hsel = jnp.where(idxm == float(t), h, hsel)
    out_ref[...] = hsel


def _proj_body(e_ref, g_ref, c_ref, o_ref, we_ref, wg_ref, wc_ref, wo_ref,
               b_ref, out_ref):
    acc = jnp.dot(e_ref[...].astype(_BF16), we_ref[...],
                  preferred_element_type=_F32)
    acc = acc + jnp.dot(g_ref[...].astype(_BF16), wg_ref[...],
                        preferred_element_type=_F32)
    acc = acc + jnp.dot(c_ref[...].astype(_BF16), wc_ref[...],
                        preferred_element_type=_F32)
    acc = acc + jnp.dot(o_ref[...].astype(_BF16), wo_ref[...],
                        preferred_element_type=_F32)
    out_ref[...] = (acc + b_ref[...]).astype(out_ref.dtype)


def _rec_body(gif_ref, gib_ref, whf_ref, whb_ref, bhf_ref, bhb_ref,
              out_ref, hf_ref, hb_ref):
    t = pl.program_id(0)
    B = out_ref.shape[0]

    @pl.when(t == 0)
    def _():
        hf_ref[...] = jnp.zeros_like(hf_ref)
        hb_ref[...] = jnp.zeros_like(hb_ref)

    H = whf_ref.shape[0]

    def step(gi, h, whhT, bhh):
        gh = jnp.dot(h.astype(_BF16), whhT, preferred_element_type=_F32) + bhh
        r = jax.nn.sigmoid(gi[:, :H] + gh[:, :H])
        z = jax.nn.sigmoid(gi[:, H:2 * H] + gh[:, H:2 * H])
        n = jnp.tanh(gi[:, 2 * H:] + r * gh[:, 2 * H:])
        return (1.0 - z) * n + z * h

    hf = step(gif_ref[0].astype(_F32), hf_ref[...], whf_ref[...], bhf_ref[...])
    hf_ref[...] = hf
    out_ref[pl.ds(t, 1), :, :H] = hf[None]
    hb = step(gib_ref[0].astype(_F32), hb_ref[...], whb_ref[...], bhb_ref[...])
    hb_ref[...] = hb
    out_ref[pl.ds(B - 1 - t, 1), :, H:] = hb[None]


def kernel(char_encoded, C_lengths, elmo_embeddings, glove_embeddings,
           onehot_embeddings, cemb, c_Wih, c_Whh, c_bih, c_bhh,
           w_Wih_f, w_Whh_f, w_bih_f, w_bhh_f,
           w_Wih_b, w_Whh_b, w_bih_b, w_bhh_b):
    NW, L = char_encoded.shape
    NC, CD = cemb.shape
    B, S, DE = elmo_embeddings.shape
    DG = glove_embeddings.shape[2]
    DO = onehot_embeddings.shape[2]
    H = w_Whh_f.shape[1]

    WB = 1024
    RB = 512

    idx = jnp.clip(C_lengths - 1, 0, L - 1).astype(_F32)
    idxm = jnp.broadcast_to(idx[:, None], (NW, CD))

    char_emb = pl.pallas_call(
        _char_body,
        grid=(NW // WB,),
        in_specs=[
            pl.BlockSpec((WB, L), lambda i: (i, 0)),
            pl.BlockSpec((WB, CD), lambda i: (i, 0)),
            pl.BlockSpec((NC, CD), lambda i: (0, 0)),
            pl.BlockSpec((CD, 3 * CD), lambda i: (0, 0)),
            pl.BlockSpec((CD, 3 * CD), lambda i: (0, 0)),
            pl.BlockSpec((1, 3 * CD), lambda i: (0, 0)),
            pl.BlockSpec((1, 3 * CD), lambda i: (0, 0)),
        ],
        out_specs=pl.BlockSpec((WB, CD), lambda i: (i, 0)),
        out_shape=jax.ShapeDtypeStruct((NW, CD), _F32),
    )(char_encoded.astype(jnp.int32), idxm, cemb, c_Wih.T,
      c_Whh.T.astype(_BF16), c_bih.reshape(1, -1), c_bhh.reshape(1, -1))

    WT = jnp.concatenate([w_Wih_f.T, w_Wih_b.T], axis=1).astype(_BF16)
    WT_e = WT[:DE]
    WT_g = WT[DE:DE + DG]
    WT_c = WT[DE + DG:DE + DG + CD]
    WT_o = WT[DE + DG + CD:]
    bi = jnp.concatenate([w_bih_f, w_bih_b]).reshape(1, -1)
    G = 6 * H

    gi = pl.pallas_call(
        _proj_body,
        grid=(NW // RB,),
        in_specs=[
            pl.BlockSpec((RB, DE), lambda i: (i, 0)),
            pl.BlockSpec((RB, DG), lambda i: (i, 0)),
            pl.BlockSpec((RB, CD), lambda i: (i, 0)),
            pl.BlockSpec((RB, DO), lambda i: (i, 0)),
            pl.BlockSpec((DE, G), lambda i: (0, 0)),
            pl.BlockSpec((DG, G), lambda i: (0, 0)),
            pl.BlockSpec((CD, G), lambda i: (0, 0)),
            pl.BlockSpec((DO, G), lambda i: (0, 0)),
            pl.BlockSpec((1, G), lambda i: (0, 0)),
        ],
        out_specs=pl.BlockSpec((RB, G), lambda i: (i, 0)),
        out_shape=jax.ShapeDtypeStruct((NW, G), _BF16),
    )(elmo_embeddings.reshape(NW, DE), glove_embeddings.reshape(NW, DG),
      char_emb, onehot_embeddings.reshape(NW, DO),
      WT_e, WT_g, WT_c, WT_o, bi)

    gi3 = gi.reshape(B, S, G)

    out = pl.pallas_call(
        _rec_body,
        grid=(B,),
        in_specs=[
            pl.BlockSpec((1, S, 3 * H), lambda t: (t, 0, 0)),
            pl.BlockSpec((1, S, 3 * H), lambda t: (B - 1 - t, 0, 1)),
            pl.BlockSpec((H, 3 * H), lambda t: (0, 0)),
            pl.BlockSpec((H, 3 * H), lambda t: (0, 0)),
            pl.BlockSpec((1, 3 * H), lambda t: (0, 0)),
            pl.BlockSpec((1, 3 * H), lambda t: (0, 0)),
        ],
        out_specs=pl.BlockSpec((B, S, 2 * H), lambda t: (0, 0, 0)),
        out_shape=jax.ShapeDtypeStruct((B, S, 2 * H), _F32),
        scratch_shapes=[pltpu.VMEM((S, H), _F32), pltpu.VMEM((S, H), _F32)],
    )(gi3, gi3, w_Whh_f.T.astype(_BF16), w_Whh_b.T.astype(_BF16),
      w_bhh_f.reshape(1, -1), w_bhh_b.reshape(1, -1))

    return out


# speedup vs baseline: 4.8476x; 1.0897x over previous
import jax
import jax.numpy as jnp
from jax.experimental import pallas as pl
from jax.experimental.pallas import tpu as pltpu

_F32 = jnp.float32
_BF16 = jnp.bfloat16


def _char_body(ids_ref, idxm_ref, cemb_ref, wihT_ref, whhT_ref, bih_ref,
               bhh_ref, out_ref, gis_ref):
    WB, L = ids_ref.shape
    NC, CD = cemb_ref.shape
    E = jnp.dot(cemb_ref[...], wihT_ref[...],
                preferred_element_type=_F32).astype(_BF16)
    bih = bih_ref[...].astype(_BF16)
    bhh = bhh_ref[...].astype(_BF16)
    whhT = whhT_ref[...]
    idxm = idxm_ref[...]
    iota = jax.lax.broadcasted_iota(jnp.int32, (WB, NC), 1)
    for t in range(L):
        oh = (ids_ref[:, t:t + 1] == iota).astype(_BF16)
        gis_ref[t] = (jnp.dot(oh, E, preferred_element_type=_F32)
                      ).astype(_BF16) + bih
    h = jnp.zeros((WB, CD), _BF16)
    hsel = jnp.zeros((WB, CD), _BF16)
    for t in range(L):
        gi = gis_ref[t]
        gh = (jnp.dot(h, whhT, preferred_element_type=_F32)
              ).astype(_BF16) + bhh
        r = jax.nn.sigmoid(gi[:, :CD] + gh[:, :CD])
        z = jax.nn.sigmoid(gi[:, CD:2 * CD] + gh[:, CD:2 * CD])
        n = jnp.tanh(gi[:, 2 * CD:] + r * gh[:, 2 * CD:])
        h = (1.0 - z) * n + z * h
        hsel = jnp.where(idxm == float(t), h, hsel)
    out_ref[...] = hsel.astype(_F32)


def _proj_body(e_ref, g_ref, c_ref, o_ref, we_ref, wg_ref, wc_ref, wo_ref,
               b_ref, out_ref):
    acc = jnp.dot(e_ref[...].astype(_BF16), we_ref[...],
                  preferred_element_type=_F32)
    acc = acc + jnp.dot(g_ref[...].astype(_BF16), wg_ref[...],
                        preferred_element_type=_F32)
    acc = acc + jnp.dot(c_ref[...].astype(_BF16), wc_ref[...],
                        preferred_element_type=_F32)
    acc = acc + jnp.dot(o_ref[...].astype(_BF16), wo_ref[...],
                        preferred_element_type=_F32)
    out_ref[...] = (acc + b_ref[...]).astype(out_ref.dtype)


def _rec_body(gif_ref, gib_ref, whf_ref, whb_ref, bhf_ref, bhb_ref,
              out_ref, hf_ref, hb_ref):
    c = pl.program_id(0)
    B = out_ref.shape[0]
    TS = gif_ref.shape[0]

    @pl.when(c == 0)
    def _():
        hf_ref[...] = jnp.zeros_like(hf_ref)
        hb_ref[...] = jnp.zeros_like(hb_ref)

    H = whf_ref.shape[0]

    def step(gi, h, whhT, bhh):
        gh = jnp.dot(h.astype(_BF16), whhT, preferred_element_type=_F32) + bhh
        r = jax.nn.sigmoid(gi[:, :H] + gh[:, :H])
        z = jax.nn.sigmoid(gi[:, H:2 * H] + gh[:, H:2 * H])
        n = jnp.tanh(gi[:, 2 * H:] + r * gh[:, 2 * H:])
        return (1.0 - z) * n + z * h

    hf = hf_ref[...]
    hb = hb_ref[...]
    for i in range(TS):
        hf = step(gif_ref[i].astype(_F32), hf, whf_ref[...], bhf_ref[...])
        out_ref[pl.ds(c * TS + i, 1), :, :H] = hf[None]
        hb = step(gib_ref[TS - 1 - i].astype(_F32), hb, whb_ref[...],
                  bhb_ref[...])
        out_ref[pl.ds(B - 1 - c * TS - i, 1), :, H:] = hb[None]
    hf_ref[...] = hf
    hb_ref[...] = hb


def kernel(char_encoded, C_lengths, elmo_embeddings, glove_embeddings,
           onehot_embeddings, cemb, c_Wih, c_Whh, c_bih, c_bhh,
           w_Wih_f, w_Whh_f, w_bih_f, w_bhh_f,
           w_Wih_b, w_Whh_b, w_bih_b, w_bhh_b):
    NW, L = char_encoded.shape
    NC, CD = cemb.shape
    B, S, DE = elmo_embeddings.shape
    DG = glove_embeddings.shape[2]
    DO = onehot_embeddings.shape[2]
    H = w_Whh_f.shape[1]

    WB = 1024
    RB = 1024
    TS = 8

    idx = jnp.clip(C_lengths - 1, 0, L - 1).astype(_F32)
    idxm = jnp.broadcast_to(idx[:, None], (NW, CD))

    char_emb = pl.pallas_call(
        _char_body,
        grid=(NW // WB,),
        in_specs=[
            pl.BlockSpec((WB, L), lambda i: (i, 0)),
            pl.BlockSpec((WB, CD), lambda i: (i, 0)),
            pl.BlockSpec((NC, CD), lambda i: (0, 0)),
            pl.BlockSpec((CD, 3 * CD), lambda i: (0, 0)),
            pl.BlockSpec((CD, 3 * CD), lambda i: (0, 0)),
            pl.BlockSpec((1, 3 * CD), lambda i: (0, 0)),
            pl.BlockSpec((1, 3 * CD), lambda i: (0, 0)),
        ],
        out_specs=pl.BlockSpec((WB, CD), lambda i: (i, 0)),
        out_shape=jax.ShapeDtypeStruct((NW, CD), _F32),
        scratch_shapes=[pltpu.VMEM((L, WB, 3 * CD), _BF16)],
    )(char_encoded.astype(jnp.int32), idxm, cemb, c_Wih.T,
      c_Whh.T.astype(_BF16), c_bih.reshape(1, -1), c_bhh.reshape(1, -1))

    WT = jnp.concatenate([w_Wih_f.T, w_Wih_b.T], axis=1).astype(_BF16)
    WT_e = WT[:DE]
    WT_g = WT[DE:DE + DG]
    WT_c = WT[DE + DG:DE + DG + CD]
    WT_o = WT[DE + DG + CD:]
    bi = jnp.concatenate([w_bih_f, w_bih_b]).reshape(1, -1)
    G = 6 * H

    gi = pl.pallas_call(
        _proj_body,
        grid=(NW // RB,),
        in_specs=[
            pl.BlockSpec((RB, DE), lambda i: (i, 0)),
            pl.BlockSpec((RB, DG), lambda i: (i, 0)),
            pl.BlockSpec((RB, CD), lambda i: (i, 0)),
            pl.BlockSpec((RB, DO), lambda i: (i, 0)),
            pl.BlockSpec((DE, G), lambda i: (0, 0)),
            pl.BlockSpec((DG, G), lambda i: (0, 0)),
            pl.BlockSpec((CD, G), lambda i: (0, 0)),
            pl.BlockSpec((DO, G), lambda i: (0, 0)),
            pl.BlockSpec((1, G), lambda i: (0, 0)),
        ],
        out_specs=pl.BlockSpec((RB, G), lambda i: (i, 0)),
        out_shape=jax.ShapeDtypeStruct((NW, G), _BF16),
    )(elmo_embeddings.reshape(NW, DE), glove_embeddings.reshape(NW, DG),
      char_emb, onehot_embeddings.reshape(NW, DO),
      WT_e, WT_g, WT_c, WT_o, bi)

    gi3 = gi.reshape(B, S, G)
    NT = B // TS

    out = pl.pallas_call(
        _rec_body,
        grid=(NT,),
        in_specs=[
            pl.BlockSpec((TS, S, 3 * H), lambda c: (c, 0, 0)),
            pl.BlockSpec((TS, S, 3 * H), lambda c: (NT - 1 - c, 0, 1)),
            pl.BlockSpec((H, 3 * H), lambda c: (0, 0)),
            pl.BlockSpec((H, 3 * H), lambda c: (0, 0)),
            pl.BlockSpec((1, 3 * H), lambda c: (0, 0)),
            pl.BlockSpec((1, 3 * H), lambda c: (0, 0)),
        ],
        out_specs=pl.BlockSpec((B, S, 2 * H), lambda c: (0, 0, 0)),
        out_shape=jax.ShapeDtypeStruct((B, S, 2 * H), _F32),
        scratch_shapes=[pltpu.VMEM((S, H), _F32), pltpu.VMEM((S, H), _F32)],
    )(gi3, gi3, w_Whh_f.T.astype(_BF16), w_Whh_b.T.astype(_BF16),
      w_bhh_f.reshape(1, -1), w_bhh_b.reshape(1, -1))

    return out


# speedup vs baseline: 5.2769x; 1.0886x over previous
import jax
import jax.numpy as jnp
from jax.experimental import pallas as pl
from jax.experimental.pallas import tpu as pltpu

_F32 = jnp.float32
_BF16 = jnp.bfloat16


def _charproj_body(ids_ref, idxm_ref, cemb_ref, wihT_ref, whhT_ref, bih_ref,
                   bhh_ref, e_ref, g_ref, o_ref, we_ref, wg_ref, wc_ref,
                   wo_ref, b_ref, out_ref, gis_ref):
    WB, L = ids_ref.shape
    NC, CD = cemb_ref.shape
    E = jnp.dot(cemb_ref[...], wihT_ref[...],
                preferred_element_type=_F32).astype(_BF16)
    bih = bih_ref[...].astype(_BF16)
    bhh = bhh_ref[...].astype(_BF16)
    whhT = whhT_ref[...]
    idxm = idxm_ref[...]
    iota = jax.lax.broadcasted_iota(jnp.int32, (WB, NC), 1)
    for t in range(L):
        oh = (ids_ref[:, t:t + 1] == iota).astype(_BF16)
        gis_ref[t] = (jnp.dot(oh, E, preferred_element_type=_F32)
                      ).astype(_BF16) + bih
    h = jnp.zeros((WB, CD), _BF16)
    hsel = jnp.zeros((WB, CD), _BF16)
    for t in range(L):
        gi = gis_ref[t]
        gh = (jnp.dot(h, whhT, preferred_element_type=_F32)
              ).astype(_BF16) + bhh
        r = jax.nn.sigmoid(gi[:, :CD] + gh[:, :CD])
        z = jax.nn.sigmoid(gi[:, CD:2 * CD] + gh[:, CD:2 * CD])
        n = jnp.tanh(gi[:, 2 * CD:] + r * gh[:, 2 * CD:])
        h = (1.0 - z) * n + z * h
        hsel = jnp.where(idxm == float(t), h, hsel)
    acc = jnp.dot(e_ref[...].astype(_BF16), we_ref[...],
                  preferred_element_type=_F32)
    acc = acc + jnp.dot(g_ref[...].astype(_BF16), wg_ref[...],
                        preferred_element_type=_F32)
    acc = acc + jnp.dot(o_ref[...].astype(_BF16), wo_ref[...],
                        preferred_element_type=_F32)
    acc = acc + jnp.dot(hsel, wc_ref[...], preferred_element_type=_F32)
    out_ref[...] = (acc + b_ref[...]).astype(out_ref.dtype)


def _rec_body(gif_ref, gib_ref, whf_ref, whb_ref, bhf_ref, bhb_ref,
              out_ref, hf_ref, hb_ref):
    c = pl.program_id(0)
    B = out_ref.shape[0]
    TS = gif_ref.shape[0]

    @pl.when(c == 0)
    def _():
        hf_ref[...] = jnp.zeros_like(hf_ref)
        hb_ref[...] = jnp.zeros_like(hb_ref)

    H = whf_ref.shape[0]

    def step(gi, h, whhT, bhh):
        gh = jnp.dot(h.astype(_BF16), whhT, preferred_element_type=_F32) + bhh
        r = jax.nn.sigmoid(gi[:, :H] + gh[:, :H])
        z = jax.nn.sigmoid(gi[:, H:2 * H] + gh[:, H:2 * H])
        n = jnp.tanh(gi[:, 2 * H:] + r * gh[:, 2 * H:])
        return (1.0 - z) * n + z * h

    hf = hf_ref[...]
    hb = hb_ref[...]
    for i in range(TS):
        hf = step(gif_ref[i].astype(_F32), hf, whf_ref[...], bhf_ref[...])
        out_ref[pl.ds(c * TS + i, 1), :, :H] = hf[None]
        hb = step(gib_ref[TS - 1 - i].astype(_F32), hb, whb_ref[...],
                  bhb_ref[...])
        out_ref[pl.ds(B - 1 - c * TS - i, 1), :, H:] = hb[None]
    hf_ref[...] = hf
    hb_ref[...] = hb


def kernel(char_encoded, C_lengths, elmo_embeddings, glove_embeddings,
           onehot_embeddings, cemb, c_Wih, c_Whh, c_bih, c_bhh,
           w_Wih_f, w_Whh_f, w_bih_f, w_bhh_f,
           w_Wih_b, w_Whh_b, w_bih_b, w_bhh_b):
    NW, L = char_encoded.shape
    NC, CD = cemb.shape
    B, S, DE = elmo_embeddings.shape
    DG = glove_embeddings.shape[2]
    DO = onehot_embeddings.shape[2]
    H = w_Whh_f.shape[1]

    WB = 1024
    TS = 8

    idx = jnp.clip(C_lengths - 1, 0, L - 1).astype(_F32)
    idxm = jnp.broadcast_to(idx[:, None], (NW, CD))

    WT = jnp.concatenate([w_Wih_f.T, w_Wih_b.T], axis=1).astype(_BF16)
    WT_e = WT[:DE]
    WT_g = WT[DE:DE + DG]
    WT_c = WT[DE + DG:DE + DG + CD]
    WT_o = WT[DE + DG + CD:]
    bi = jnp.concatenate([w_bih_f, w_bih_b]).reshape(1, -1)
    G = 6 * H

    gi = pl.pallas_call(
        _charproj_body,
        grid=(NW // WB,),
        in_specs=[
            pl.BlockSpec((WB, L), lambda i: (i, 0)),
            pl.BlockSpec((WB, CD), lambda i: (i, 0)),
            pl.BlockSpec((NC, CD), lambda i: (0, 0)),
            pl.BlockSpec((CD, 3 * CD), lambda i: (0, 0)),
            pl.BlockSpec((CD, 3 * CD), lambda i: (0, 0)),
            pl.BlockSpec((1, 3 * CD), lambda i: (0, 0)),
            pl.BlockSpec((1, 3 * CD), lambda i: (0, 0)),
            pl.BlockSpec((WB, DE), lambda i: (i, 0)),
            pl.BlockSpec((WB, DG), lambda i: (i, 0)),
            pl.BlockSpec((WB, DO), lambda i: (i, 0)),
            pl.BlockSpec((DE, G), lambda i: (0, 0)),
            pl.BlockSpec((DG, G), lambda i: (0, 0)),
            pl.BlockSpec((CD, G), lambda i: (0, 0)),
            pl.BlockSpec((DO, G), lambda i: (0, 0)),
            pl.BlockSpec((1, G), lambda i: (0, 0)),
        ],
        out_specs=pl.BlockSpec((WB, G), lambda i: (i, 0)),
        out_shape=jax.ShapeDtypeStruct((NW, G), _BF16),
        scratch_shapes=[pltpu.VMEM((L, WB, 3 * CD), _BF16)],
    )(char_encoded.astype(jnp.int32), idxm, cemb, c_Wih.T,
      c_Whh.T.astype(_BF16), c_bih.reshape(1, -1), c_bhh.reshape(1, -1),
      elmo_embeddings.reshape(NW, DE), glove_embeddings.reshape(NW, DG),
      onehot_embeddings.reshape(NW, DO), WT_e, WT_g, WT_c, WT_o, bi)

    gi3 = gi.reshape(B, S, G)
    NT = B // TS

    out = pl.pallas_call(
        _rec_body,
        grid=(NT,),
        in_specs=[
            pl.BlockSpec((TS, S, 3 * H), lambda c: (c, 0, 0)),
            pl.BlockSpec((TS, S, 3 * H), lambda c: (NT - 1 - c, 0, 1)),
            pl.BlockSpec((H, 3 * H), lambda c: (0, 0)),
            pl.BlockSpec((H, 3 * H), lambda c: (0, 0)),
            pl.BlockSpec((1, 3 * H), lambda c: (0, 0)),
            pl.BlockSpec((1, 3 * H), lambda c: (0, 0)),
        ],
        out_specs=pl.BlockSpec((B, S, 2 * H), lambda c: (0, 0, 0)),
        out_shape=jax.ShapeDtypeStruct((B, S, 2 * H), _F32),
        scratch_shapes=[pltpu.VMEM((S, H), _F32), pltpu.VMEM((S, H), _F32)],
    )(gi3, gi3, w_Whh_f.T.astype(_BF16), w_Whh_b.T.astype(_BF16),
      w_bhh_f.reshape(1, -1), w_bhh_b.reshape(1, -1))

    return out


# speedup vs baseline: 6.5021x; 1.2322x over previous
import jax
import jax.numpy as jnp
from jax.experimental import pallas as pl
from jax.experimental.pallas import tpu as pltpu

_F32 = jnp.float32
_BF16 = jnp.bfloat16


def _charproj_body(ids_ref, idxm_ref, cemb_ref, wihT_ref, whhT_ref, bih_ref,
                   bhh_ref, e_ref, g_ref, o_ref, we_ref, wg_ref, wc_ref,
                   wo_ref, b_ref, out_ref):
    WB, L = ids_ref.shape
    NC, CD = cemb_ref.shape
    E = jnp.dot(cemb_ref[...], wihT_ref[...],
                preferred_element_type=_F32).astype(_BF16)
    bih = bih_ref[...].astype(_BF16)
    bhh = bhh_ref[...].astype(_BF16)
    whhT = whhT_ref[...]
    idxm = idxm_ref[...]
    iota = jax.lax.broadcasted_iota(jnp.int32, (WB, NC), 1).astype(_BF16)
    h = jnp.zeros((WB, CD), _BF16)
    hsel = jnp.zeros((WB, CD), _BF16)
    for t in range(L):
        oh = (ids_ref[:, t:t + 1] == iota).astype(_BF16)
        gi = (jnp.dot(oh, E, preferred_element_type=_F32)
              ).astype(_BF16) + bih
        gh = (jnp.dot(h, whhT, preferred_element_type=_F32)
              ).astype(_BF16) + bhh
        r = jax.nn.sigmoid(gi[:, :CD] + gh[:, :CD])
        z = jax.nn.sigmoid(gi[:, CD:2 * CD] + gh[:, CD:2 * CD])
        n = jnp.tanh(gi[:, 2 * CD:] + r * gh[:, 2 * CD:])
        h = (1.0 - z) * n + z * h
        hsel = jnp.where(idxm == float(t), h, hsel)
    acc = jnp.dot(e_ref[...].astype(_BF16), we_ref[...],
                  preferred_element_type=_F32)
    acc = acc + jnp.dot(g_ref[...].astype(_BF16), wg_ref[...],
                        preferred_element_type=_F32)
    acc = acc + jnp.dot(o_ref[...].astype(_BF16), wo_ref[...],
                        preferred_element_type=_F32)
    acc = acc + jnp.dot(hsel, wc_ref[...], preferred_element_type=_F32)
    out_ref[...] = (acc + b_ref[...]).astype(out_ref.dtype)


def _rec_body(gif_ref, gib_ref, whf_ref, whb_ref, bhf_ref, bhb_ref,
              out_ref, hf_ref, hb_ref):
    c = pl.program_id(0)
    B = out_ref.shape[0]
    TS = gif_ref.shape[0]

    @pl.when(c == 0)
    def _():
        hf_ref[...] = jnp.zeros_like(hf_ref)
        hb_ref[...] = jnp.zeros_like(hb_ref)

    H = whf_ref.shape[0]

    def step(gi, h, whhT, bhh):
        gh = jnp.dot(h.astype(_BF16), whhT, preferred_element_type=_F32) + bhh
        r = jax.nn.sigmoid(gi[:, :H] + gh[:, :H])
        z = jax.nn.sigmoid(gi[:, H:2 * H] + gh[:, H:2 * H])
        n = jnp.tanh(gi[:, 2 * H:] + r * gh[:, 2 * H:])
        return (1.0 - z) * n + z * h

    hf = hf_ref[...]
    hb = hb_ref[...]
    for i in range(TS):
        hf = step(gif_ref[i].astype(_F32), hf, whf_ref[...], bhf_ref[...])
        out_ref[pl.ds(c * TS + i, 1), :, :H] = hf[None]
        hb = step(gib_ref[TS - 1 - i].astype(_F32), hb, whb_ref[...],
                  bhb_ref[...])
        out_ref[pl.ds(B - 1 - c * TS - i, 1), :, H:] = hb[None]
    hf_ref[...] = hf
    hb_ref[...] = hb


def kernel(char_encoded, C_lengths, elmo_embeddings, glove_embeddings,
           onehot_embeddings, cemb, c_Wih, c_Whh, c_bih, c_bhh,
           w_Wih_f, w_Whh_f, w_bih_f, w_bhh_f,
           w_Wih_b, w_Whh_b, w_bih_b, w_bhh_b):
    NW, L = char_encoded.shape
    NC, CD = cemb.shape
    B, S, DE = elmo_embeddings.shape
    DG = glove_embeddings.shape[2]
    DO = onehot_embeddings.shape[2]
    H = w_Whh_f.shape[1]

    WB = 1024
    TS = 8

    idx = jnp.clip(C_lengths - 1, 0, L - 1).astype(_BF16)
    idxm = jnp.broadcast_to(idx[:, None], (NW, CD))

    WT = jnp.concatenate([w_Wih_f.T, w_Wih_b.T], axis=1).astype(_BF16)
    WT_e = WT[:DE]
    WT_g = WT[DE:DE + DG]
    WT_c = WT[DE + DG:DE + DG + CD]
    WT_o = WT[DE + DG + CD:]
    bi = jnp.concatenate([w_bih_f, w_bih_b]).reshape(1, -1)
    G = 6 * H

    gi = pl.pallas_call(
        _charproj_body,
        grid=(NW // WB,),
        in_specs=[
            pl.BlockSpec((WB, L), lambda i: (i, 0)),
            pl.BlockSpec((WB, CD), lambda i: (i, 0)),
            pl.BlockSpec((NC, CD), lambda i: (0, 0)),
            pl.BlockSpec((CD, 3 * CD), lambda i: (0, 0)),
            pl.BlockSpec((CD, 3 * CD), lambda i: (0, 0)),
            pl.BlockSpec((1, 3 * CD), lambda i: (0, 0)),
            pl.BlockSpec((1, 3 * CD), lambda i: (0, 0)),
            pl.BlockSpec((WB, DE), lambda i: (i, 0)),
            pl.BlockSpec((WB, DG), lambda i: (i, 0)),
            pl.BlockSpec((WB, DO), lambda i: (i, 0)),
            pl.BlockSpec((DE, G), lambda i: (0, 0)),
            pl.BlockSpec((DG, G), lambda i: (0, 0)),
            pl.BlockSpec((CD, G), lambda i: (0, 0)),
            pl.BlockSpec((DO, G), lambda i: (0, 0)),
            pl.BlockSpec((1, G), lambda i: (0, 0)),
        ],
        out_specs=pl.BlockSpec((WB, G), lambda i: (i, 0)),
        out_shape=jax.ShapeDtypeStruct((NW, G), _BF16),
    )(char_encoded.astype(_BF16), idxm, cemb, c_Wih.T,
      c_Whh.T.astype(_BF16), c_bih.reshape(1, -1), c_bhh.reshape(1, -1),
      elmo_embeddings.reshape(NW, DE), glove_embeddings.reshape(NW, DG),
      onehot_embeddings.reshape(NW, DO), WT_e, WT_g, WT_c, WT_o, bi)

    gi3 = gi.reshape(B, S, G)
    NT = B // TS

    out = pl.pallas_call(
        _rec_body,
        grid=(NT,),
        in_specs=[
            pl.BlockSpec((TS, S, 3 * H), lambda c: (c, 0, 0)),
            pl.BlockSpec((TS, S, 3 * H), lambda c: (NT - 1 - c, 0, 1)),
            pl.BlockSpec((H, 3 * H), lambda c: (0, 0)),
            pl.BlockSpec((H, 3 * H), lambda c: (0, 0)),
            pl.BlockSpec((1, 3 * H), lambda c: (0, 0)),
            pl.BlockSpec((1, 3 * H), lambda c: (0, 0)),
        ],
        out_specs=pl.BlockSpec((B, S, 2 * H), lambda c: (0, 0, 0)),
        out_shape=jax.ShapeDtypeStruct((B, S, 2 * H), _F32),
        scratch_shapes=[pltpu.VMEM((S, H), _F32), pltpu.VMEM((S, H), _F32)],
    )(gi3, gi3, w_Whh_f.T.astype(_BF16), w_Whh_b.T.astype(_BF16),
      w_bhh_f.reshape(1, -1), w_bhh_b.reshape(1, -1))

    return out
